# Initial kernel scaffold; baseline (speedup 1.0000x reference)
import jax
import jax.numpy as jnp
from jax.experimental import pallas as pl


def kernel(x, edge_index, batch, W_rel1, b_rel1, W_root1, W_rel2, b_rel2, W_root2, Wd, bd):
    raise NotImplementedError("write your pallas kernel here")


# speedup vs baseline: 5.8454x; 5.8454x over previous
import functools

import jax
import jax.numpy as jnp
from jax import lax
from jax.experimental import pallas as pl
from jax.experimental.pallas import tpu as pltpu
from jax.experimental.pallas import tpu_sc as plsc

N = 10000
E = 320000
D = 128
G = 256
K = 10

NC = 2
NS = 16
NW = NC * NS
CH = 128
NCHUNK = E // CH
BASE_CH = NCHUNK // NW
EXTRA_CH = NCHUNK - BASE_CH * NW
RPT = 632
RPT_LAST = N - RPT * (NS - 1)


def _segsum_body(x_hbm, src_hbm, dst_hbm, zero_hbm, out_hbm,
                 src_v, dst_v, rows_v, acc_sh, sem):
    c = lax.axis_index("c")
    s = lax.axis_index("s")
    wid = s * NC + c

    @pl.when(s == 0)
    def _():
        pltpu.sync_copy(zero_hbm, acc_sh)

    plsc.subcore_barrier()

    nch = jnp.where(wid < EXTRA_CH, BASE_CH + 1, BASE_CH)

    def body(i, carry):
        off = (wid + i * NW) * CH
        pltpu.sync_copy(src_hbm.at[pl.ds(off, CH)], src_v)
        pltpu.sync_copy(dst_hbm.at[pl.ds(off, CH)], dst_v)
        pltpu.async_copy(x_hbm.at[src_v], rows_v, sem).wait()
        pltpu.sync_copy(rows_v, acc_sh.at[dst_v], add=True)
        return carry

    lax.fori_loop(0, nch, body, 0)

    plsc.subcore_barrier()

    @pl.when(s < NS - 1)
    def _():
        pltpu.sync_copy(acc_sh.at[pl.ds(s * RPT, RPT)],
                        out_hbm.at[c, pl.ds(s * RPT, RPT)])

    @pl.when(s == NS - 1)
    def _():
        pltpu.sync_copy(acc_sh.at[pl.ds((NS - 1) * RPT, RPT_LAST)],
                        out_hbm.at[c, pl.ds((NS - 1) * RPT, RPT_LAST)])


@functools.lru_cache(maxsize=1)
def _make_sc_segsum():
    return functools.partial(
        pl.kernel,
        out_type=jax.ShapeDtypeStruct((NC, N, D), jnp.float32),
        mesh=plsc.VectorSubcoreMesh(core_axis_name="c", subcore_axis_name="s"),
        scratch_types=[
            pltpu.VMEM((CH,), jnp.int32),
            pltpu.VMEM((CH,), jnp.int32),
            pltpu.VMEM((CH, D), jnp.float32),
            pltpu.VMEM_SHARED((N, D), jnp.float32),
            pltpu.SemaphoreType.DMA,
        ],
    )(_segsum_body)


R = 2000
NB = N // R


def _mm(a, b):
    return lax.dot_general(a, b, (((1,), (0,)), ((), ())),
                           preferred_element_type=jnp.float32)


def _mm_exact(a, b, dims=(((1,), (0,)), ((), ()))):
    return lax.dot_general(a, b, dims,
                           preferred_element_type=jnp.float32,
                           precision=lax.Precision.HIGHEST)


def _conv1_body(agg_ref, x_ref, wrel_ref, brel_ref, wroot_ref, o_ref):
    a = agg_ref[0] + agg_ref[1]
    h = _mm(a, wrel_ref[...]) + brel_ref[...] + _mm(x_ref[...], wroot_ref[...])
    o_ref[...] = jnp.maximum(h, 0.0)


def _tc_conv1(aggp, x, wrel, brel, wroot):
    return pl.pallas_call(
        _conv1_body,
        grid=(NB,),
        in_specs=[
            pl.BlockSpec((NC, R, D), lambda i: (0, i, 0)),
            pl.BlockSpec((R, D), lambda i: (i, 0)),
            pl.BlockSpec((D, D), lambda i: (0, 0)),
            pl.BlockSpec((1, D), lambda i: (0, 0)),
            pl.BlockSpec((D, D), lambda i: (0, 0)),
        ],
        out_specs=pl.BlockSpec((R, D), lambda i: (i, 0)),
        out_shape=jax.ShapeDtypeStruct((N, D), jnp.float32),
    )(aggp, x, wrel, brel, wroot)


def _final_body(agg_ref, h1_ref, wrel_ref, brel_ref, wroot_ref,
                batch_ref, wd_ref, bd_ref, o_ref, psum, cnt):
    i = pl.program_id(0)

    @pl.when(i == 0)
    def _():
        psum[...] = jnp.zeros((G, D), jnp.float32)
        cnt[...] = jnp.zeros((G, D), jnp.float32)

    a = agg_ref[0] + agg_ref[1]
    h2 = _mm(a, wrel_ref[...]) + brel_ref[...] + _mm(h1_ref[...], wroot_ref[...])
    h2 = jnp.maximum(h2, 0.0)

    oh = (batch_ref[...] == lax.broadcasted_iota(jnp.int32, (R, G), 1))
    oh = oh.astype(jnp.float32)
    psum[...] += _mm_exact(oh, h2, (((0,), (0,)), ((), ())))
    cnt[...] += _mm_exact(oh, jnp.ones((R, D), jnp.float32),
                          (((0,), (0,)), ((), ())))

    @pl.when(i == NB - 1)
    def _():
        pooled = psum[...] / jnp.maximum(cnt[...], 1.0)
        pp = lax.dot_general(pooled, pooled, (((1,), (1,)), ((), ())),
                             preferred_element_type=jnp.float32)
        r2 = pooled * pooled
        sq_row = _mm_exact(jnp.ones((1, D), jnp.float32), r2,
                           (((1,), (1,)), ((), ())))
        d2 = sq_row - 2.0 * pp

        iota_c = lax.broadcasted_iota(jnp.int32, (G, G), 1)
        out = jnp.full((G, D), -jnp.inf, jnp.float32)
        d2w = d2
        for _ in range(K):
            m = jnp.min(d2w, axis=1, keepdims=True)
            am = jnp.min(jnp.where(d2w == m, iota_c, G), axis=1, keepdims=True)
            sel = (iota_c == am).astype(jnp.float32)
            xj = _mm_exact(sel, pooled)
            cat = jnp.concatenate([pooled, xj - pooled], axis=1)
            msg = _mm(cat, wd_ref[...]) + bd_ref[...]
            out = jnp.maximum(out, jnp.where(msg >= 0.0, msg, 0.01 * msg))
            d2w = jnp.where(iota_c == am, jnp.inf, d2w)
        o_ref[...] = out


def _tc_final(aggp, h1, wrel, brel, wroot, batch2d, wd, bd):
    return pl.pallas_call(
        _final_body,
        grid=(NB,),
        in_specs=[
            pl.BlockSpec((NC, R, D), lambda i: (0, i, 0)),
            pl.BlockSpec((R, D), lambda i: (i, 0)),
            pl.BlockSpec((D, D), lambda i: (0, 0)),
            pl.BlockSpec((1, D), lambda i: (0, 0)),
            pl.BlockSpec((D, D), lambda i: (0, 0)),
            pl.BlockSpec((R, 1), lambda i: (i, 0)),
            pl.BlockSpec((2 * D, D), lambda i: (0, 0)),
            pl.BlockSpec((1, D), lambda i: (0, 0)),
        ],
        out_specs=pl.BlockSpec((G, D), lambda i: (0, 0)),
        out_shape=jax.ShapeDtypeStruct((G, D), jnp.float32),
        scratch_shapes=[
            pltpu.VMEM((G, D), jnp.float32),
            pltpu.VMEM((G, D), jnp.float32),
        ],
    )(aggp, h1, wrel, brel, wroot, batch2d, wd, bd)


def kernel(x, edge_index, batch, W_rel1, b_rel1, W_root1,
           W_rel2, b_rel2, W_root2, Wd, bd):
    src = edge_index[0]
    dst = edge_index[1]
    zero = jnp.zeros((N, D), jnp.float32)
    batch2d = batch.reshape(N, 1)

    sc_segsum = _make_sc_segsum()
    aggp1 = sc_segsum(x, src, dst, zero)
    h1 = _tc_conv1(aggp1, x, W_rel1, b_rel1.reshape(1, D), W_root1)
    aggp2 = sc_segsum(h1, src, dst, zero)
    out = _tc_final(aggp2, h1, W_rel2, b_rel2.reshape(1, D), W_root2,
                    batch2d, Wd, bd.reshape(1, D))
    return out


# speedup vs baseline: 10.2126x; 1.7471x over previous
import functools

import jax
import jax.numpy as jnp
from jax import lax
from jax.experimental import pallas as pl
from jax.experimental.pallas import tpu as pltpu
from jax.experimental.pallas import tpu_sc as plsc

N = 10000
E = 320000
D = 128
G = 256
K = 10

NC = 2
NS = 16
NW = NC * NS
CH = 128
NCHUNK = E // CH
BASE_CH = NCHUNK // NW
EXTRA_CH = NCHUNK - BASE_CH * NW
RPT = 632
RPT_LAST = N - RPT * (NS - 1)


NITER = 80


def _segsum_body(x_hbm, src_hbm, dst_hbm, zero_hbm, out_hbm,
                 src_v0, src_v1, dst_v0, dst_v1, rows_v0, rows_v1,
                 acc_sh, semi0, semi1, semg0, semg1):
    c = lax.axis_index("c")
    s = lax.axis_index("s")
    wid = s * NC + c

    @pl.when(s == 0)
    def _():
        pltpu.sync_copy(zero_hbm, acc_sh)

    plsc.subcore_barrier()

    src_v = (src_v0, src_v1)
    dst_v = (dst_v0, dst_v1)
    rows_v = (rows_v0, rows_v1)
    semi = (semi0, semi1)
    semg = (semg0, semg1)

    def chunk_off(i):
        ch = wid + i * NW
        return jnp.where(ch < NCHUNK, ch, 0) * CH

    def start_idx(i, b):
        off = chunk_off(i)
        pltpu.async_copy(src_hbm.at[pl.ds(off, CH)], src_v[b], semi[b])
        pltpu.async_copy(dst_hbm.at[pl.ds(off, CH)], dst_v[b], semi[b])

    def wait_idx(i, b):
        off = chunk_off(i)
        pltpu.make_async_copy(src_hbm.at[pl.ds(off, CH)], src_v[b], semi[b]).wait()
        pltpu.make_async_copy(dst_hbm.at[pl.ds(off, CH)], dst_v[b], semi[b]).wait()

    pltpu.sync_copy(src_hbm.at[pl.ds(chunk_off(0), CH)], src_v0)
    pltpu.sync_copy(dst_hbm.at[pl.ds(chunk_off(0), CH)], dst_v0)
    pltpu.async_copy(x_hbm.at[src_v0], rows_v0, semg0)
    start_idx(1, 1)

    def pair_body(jj, carry):
        for b in (0, 1):
            i = 2 * jj + b
            nb = 1 - b

            @pl.when(i + 1 < NITER)
            def _():
                wait_idx(i + 1, nb)
                pltpu.async_copy(x_hbm.at[src_v[nb]], rows_v[nb], semg[nb])

            pltpu.make_async_copy(x_hbm.at[src_v[b]], rows_v[b], semg[b]).wait()

            @pl.when(wid + i * NW < NCHUNK)
            def _():
                pltpu.sync_copy(rows_v[b], acc_sh.at[dst_v[b]], add=True)

            @pl.when(i + 2 < NITER)
            def _():
                start_idx(i + 2, b)
        return carry

    lax.fori_loop(0, NITER // 2, pair_body, 0)

    plsc.subcore_barrier()

    @pl.when(s < NS - 1)
    def _():
        pltpu.sync_copy(acc_sh.at[pl.ds(s * RPT, RPT)],
                        out_hbm.at[c, pl.ds(s * RPT, RPT)])

    @pl.when(s == NS - 1)
    def _():
        pltpu.sync_copy(acc_sh.at[pl.ds((NS - 1) * RPT, RPT_LAST)],
                        out_hbm.at[c, pl.ds((NS - 1) * RPT, RPT_LAST)])


@functools.lru_cache(maxsize=1)
def _make_sc_segsum():
    return functools.partial(
        pl.kernel,
        out_type=jax.ShapeDtypeStruct((NC, N, D), jnp.float32),
        mesh=plsc.VectorSubcoreMesh(core_axis_name="c", subcore_axis_name="s"),
        scratch_types=[
            pltpu.VMEM((CH,), jnp.int32),
            pltpu.VMEM((CH,), jnp.int32),
            pltpu.VMEM((CH,), jnp.int32),
            pltpu.VMEM((CH,), jnp.int32),
            pltpu.VMEM((CH, D), jnp.float32),
            pltpu.VMEM((CH, D), jnp.float32),
            pltpu.VMEM_SHARED((N, D), jnp.float32),
            pltpu.SemaphoreType.DMA,
            pltpu.SemaphoreType.DMA,
            pltpu.SemaphoreType.DMA,
            pltpu.SemaphoreType.DMA,
        ],
    )(_segsum_body)


R = 2000
NB = N // R


def _mm(a, b):
    return lax.dot_general(a, b, (((1,), (0,)), ((), ())),
                           preferred_element_type=jnp.float32)


def _mm_exact(a, b, dims=(((1,), (0,)), ((), ()))):
    return lax.dot_general(a, b, dims,
                           preferred_element_type=jnp.float32,
                           precision=lax.Precision.HIGHEST)


def _conv1_body(agg_ref, x_ref, wrel_ref, brel_ref, wroot_ref, o_ref):
    a = agg_ref[0] + agg_ref[1]
    h = _mm(a, wrel_ref[...]) + brel_ref[...] + _mm(x_ref[...], wroot_ref[...])
    o_ref[...] = jnp.maximum(h, 0.0)


def _tc_conv1(aggp, x, wrel, brel, wroot):
    return pl.pallas_call(
        _conv1_body,
        grid=(NB,),
        in_specs=[
            pl.BlockSpec((NC, R, D), lambda i: (0, i, 0)),
            pl.BlockSpec((R, D), lambda i: (i, 0)),
            pl.BlockSpec((D, D), lambda i: (0, 0)),
            pl.BlockSpec((1, D), lambda i: (0, 0)),
            pl.BlockSpec((D, D), lambda i: (0, 0)),
        ],
        out_specs=pl.BlockSpec((R, D), lambda i: (i, 0)),
        out_shape=jax.ShapeDtypeStruct((N, D), jnp.float32),
    )(aggp, x, wrel, brel, wroot)


def _final_body(agg_ref, h1_ref, wrel_ref, brel_ref, wroot_ref,
                batch_ref, wd_ref, bd_ref, o_ref, psum, cnt):
    i = pl.program_id(0)

    @pl.when(i == 0)
    def _():
        psum[...] = jnp.zeros((G, D), jnp.float32)
        cnt[...] = jnp.zeros((G, D), jnp.float32)

    a = agg_ref[0] + agg_ref[1]
    h2 = _mm(a, wrel_ref[...]) + brel_ref[...] + _mm(h1_ref[...], wroot_ref[...])
    h2 = jnp.maximum(h2, 0.0)

    oh = (batch_ref[...] == lax.broadcasted_iota(jnp.int32, (R, G), 1))
    oh = oh.astype(jnp.float32)
    psum[...] += _mm_exact(oh, h2, (((0,), (0,)), ((), ())))
    cnt[...] += _mm_exact(oh, jnp.ones((R, D), jnp.float32),
                          (((0,), (0,)), ((), ())))

    @pl.when(i == NB - 1)
    def _():
        pooled = psum[...] / jnp.maximum(cnt[...], 1.0)
        pp = lax.dot_general(pooled, pooled, (((1,), (1,)), ((), ())),
                             preferred_element_type=jnp.float32)
        r2 = pooled * pooled
        sq_row = _mm_exact(jnp.ones((1, D), jnp.float32), r2,
                           (((1,), (1,)), ((), ())))
        d2 = sq_row - 2.0 * pp

        iota_c = lax.broadcasted_iota(jnp.int32, (G, G), 1)
        out = jnp.full((G, D), -jnp.inf, jnp.float32)
        d2w = d2
        for _ in range(K):
            m = jnp.min(d2w, axis=1, keepdims=True)
            am = jnp.min(jnp.where(d2w == m, iota_c, G), axis=1, keepdims=True)
            sel = (iota_c == am).astype(jnp.float32)
            xj = _mm_exact(sel, pooled)
            cat = jnp.concatenate([pooled, xj - pooled], axis=1)
            msg = _mm(cat, wd_ref[...]) + bd_ref[...]
            out = jnp.maximum(out, jnp.where(msg >= 0.0, msg, 0.01 * msg))
            d2w = jnp.where(iota_c == am, jnp.inf, d2w)
        o_ref[...] = out


def _tc_final(aggp, h1, wrel, brel, wroot, batch2d, wd, bd):
    return pl.pallas_call(
        _final_body,
        grid=(NB,),
        in_specs=[
            pl.BlockSpec((NC, R, D), lambda i: (0, i, 0)),
            pl.BlockSpec((R, D), lambda i: (i, 0)),
            pl.BlockSpec((D, D), lambda i: (0, 0)),
            pl.BlockSpec((1, D), lambda i: (0, 0)),
            pl.BlockSpec((D, D), lambda i: (0, 0)),
            pl.BlockSpec((R, 1), lambda i: (i, 0)),
            pl.BlockSpec((2 * D, D), lambda i: (0, 0)),
            pl.BlockSpec((1, D), lambda i: (0, 0)),
        ],
        out_specs=pl.BlockSpec((G, D), lambda i: (0, 0)),
        out_shape=jax.ShapeDtypeStruct((G, D), jnp.float32),
        scratch_shapes=[
            pltpu.VMEM((G, D), jnp.float32),
            pltpu.VMEM((G, D), jnp.float32),
        ],
    )(aggp, h1, wrel, brel, wroot, batch2d, wd, bd)


def kernel(x, edge_index, batch, W_rel1, b_rel1, W_root1,
           W_rel2, b_rel2, W_root2, Wd, bd):
    src = edge_index[0]
    dst = edge_index[1]
    zero = jnp.zeros((N, D), jnp.float32)
    batch2d = batch.reshape(N, 1)

    sc_segsum = _make_sc_segsum()
    aggp1 = sc_segsum(x, src, dst, zero)
    h1 = _tc_conv1(aggp1, x, W_rel1, b_rel1.reshape(1, D), W_root1)
    aggp2 = sc_segsum(h1, src, dst, zero)
    out = _tc_final(aggp2, h1, W_rel2, b_rel2.reshape(1, D), W_root2,
                    batch2d, Wd, bd.reshape(1, D))
    return out


# speedup vs baseline: 10.3655x; 1.0150x over previous
import functools

import jax
import jax.numpy as jnp
from jax import lax
from jax.experimental import pallas as pl
from jax.experimental.pallas import tpu as pltpu
from jax.experimental.pallas import tpu_sc as plsc

N = 10000
E = 320000
D = 128
G = 256
K = 10

NC = 2
NS = 16
NW = NC * NS
CH = 128
NCHUNK = E // CH
BASE_CH = NCHUNK // NW
EXTRA_CH = NCHUNK - BASE_CH * NW
RPT = 632
RPT_LAST = N - RPT * (NS - 1)


NITER = 81


NBUF = 3


def _segsum_body(x_hbm, src_hbm, dst_hbm, zero_hbm, out_hbm,
                 src_v0, src_v1, src_v2,
                 dst_v0, dst_v1, dst_v2,
                 rows_v0, rows_v1, rows_v2,
                 acc_sh,
                 semi0, semi1, semi2,
                 semg0, semg1, semg2):
    c = lax.axis_index("c")
    s = lax.axis_index("s")
    wid = s * NC + c

    @pl.when(s == 0)
    def _():
        pltpu.sync_copy(zero_hbm, acc_sh)

    plsc.subcore_barrier()

    src_v = (src_v0, src_v1, src_v2)
    dst_v = (dst_v0, dst_v1, dst_v2)
    rows_v = (rows_v0, rows_v1, rows_v2)
    semi = (semi0, semi1, semi2)
    semg = (semg0, semg1, semg2)

    def chunk_off(i):
        ch = wid + i * NW
        return jnp.where(ch < NCHUNK, ch, 0) * CH

    def start_idx(i, b):
        off = chunk_off(i)
        pltpu.async_copy(src_hbm.at[pl.ds(off, CH)], src_v[b], semi[b])
        pltpu.async_copy(dst_hbm.at[pl.ds(off, CH)], dst_v[b], semi[b])

    def wait_idx(i, b):
        off = chunk_off(i)
        pltpu.make_async_copy(src_hbm.at[pl.ds(off, CH)], src_v[b], semi[b]).wait()
        pltpu.make_async_copy(dst_hbm.at[pl.ds(off, CH)], dst_v[b], semi[b]).wait()

    for j in range(NBUF):
        start_idx(j, j)
    wait_idx(0, 0)
    pltpu.async_copy(x_hbm.at[src_v0], rows_v0, semg0)
    wait_idx(1, 1)
    pltpu.async_copy(x_hbm.at[src_v1], rows_v1, semg1)

    def tri_body(jj, carry):
        for b in range(NBUF):
            i = NBUF * jj + b
            b2 = (b + 2) % NBUF

            @pl.when(i + 2 < NITER)
            def _():
                wait_idx(i + 2, b2)
                pltpu.async_copy(x_hbm.at[src_v[b2]], rows_v[b2], semg[b2])

            pltpu.make_async_copy(x_hbm.at[src_v[b]], rows_v[b], semg[b]).wait()

            @pl.when(wid + i * NW < NCHUNK)
            def _():
                pltpu.sync_copy(rows_v[b], acc_sh.at[dst_v[b]], add=True)

            @pl.when(i + NBUF < NITER)
            def _():
                start_idx(i + NBUF, b)
        return carry

    lax.fori_loop(0, NITER // NBUF, tri_body, 0)

    plsc.subcore_barrier()

    @pl.when(s < NS - 1)
    def _():
        pltpu.sync_copy(acc_sh.at[pl.ds(s * RPT, RPT)],
                        out_hbm.at[c, pl.ds(s * RPT, RPT)])

    @pl.when(s == NS - 1)
    def _():
        pltpu.sync_copy(acc_sh.at[pl.ds((NS - 1) * RPT, RPT_LAST)],
                        out_hbm.at[c, pl.ds((NS - 1) * RPT, RPT_LAST)])


@functools.lru_cache(maxsize=1)
def _make_sc_segsum():
    return functools.partial(
        pl.kernel,
        out_type=jax.ShapeDtypeStruct((NC, N, D), jnp.float32),
        mesh=plsc.VectorSubcoreMesh(core_axis_name="c", subcore_axis_name="s"),
        scratch_types=(
            [pltpu.VMEM((CH,), jnp.int32)] * 6
            + [pltpu.VMEM((CH, D), jnp.float32)] * 3
            + [pltpu.VMEM_SHARED((N, D), jnp.float32)]
            + [pltpu.SemaphoreType.DMA] * 6
        ),
    )(_segsum_body)


R = 2000
NB = N // R


def _mm(a, b):
    return lax.dot_general(a, b, (((1,), (0,)), ((), ())),
                           preferred_element_type=jnp.float32)


def _mm_exact(a, b, dims=(((1,), (0,)), ((), ()))):
    return lax.dot_general(a, b, dims,
                           preferred_element_type=jnp.float32,
                           precision=lax.Precision.HIGHEST)


def _conv1_body(agg_ref, x_ref, wrel_ref, brel_ref, wroot_ref, o_ref):
    a = agg_ref[0] + agg_ref[1]
    h = _mm(a, wrel_ref[...]) + brel_ref[...] + _mm(x_ref[...], wroot_ref[...])
    o_ref[...] = jnp.maximum(h, 0.0)


def _tc_conv1(aggp, x, wrel, brel, wroot):
    return pl.pallas_call(
        _conv1_body,
        grid=(NB,),
        in_specs=[
            pl.BlockSpec((NC, R, D), lambda i: (0, i, 0)),
            pl.BlockSpec((R, D), lambda i: (i, 0)),
            pl.BlockSpec((D, D), lambda i: (0, 0)),
            pl.BlockSpec((1, D), lambda i: (0, 0)),
            pl.BlockSpec((D, D), lambda i: (0, 0)),
        ],
        out_specs=pl.BlockSpec((R, D), lambda i: (i, 0)),
        out_shape=jax.ShapeDtypeStruct((N, D), jnp.float32),
    )(aggp, x, wrel, brel, wroot)


def _final_body(agg_ref, h1_ref, wrel_ref, brel_ref, wroot_ref,
                batch_ref, wd_ref, bd_ref, o_ref, psum, cnt):
    i = pl.program_id(0)

    @pl.when(i == 0)
    def _():
        psum[...] = jnp.zeros((G, D), jnp.float32)
        cnt[...] = jnp.zeros((G, D), jnp.float32)

    a = agg_ref[0] + agg_ref[1]
    h2 = _mm(a, wrel_ref[...]) + brel_ref[...] + _mm(h1_ref[...], wroot_ref[...])
    h2 = jnp.maximum(h2, 0.0)

    oh = (batch_ref[...] == lax.broadcasted_iota(jnp.int32, (R, G), 1))
    oh = oh.astype(jnp.float32)
    psum[...] += _mm_exact(oh, h2, (((0,), (0,)), ((), ())))
    cnt[...] += _mm_exact(oh, jnp.ones((R, D), jnp.float32),
                          (((0,), (0,)), ((), ())))

    @pl.when(i == NB - 1)
    def _():
        pooled = psum[...] / jnp.maximum(cnt[...], 1.0)
        pp = lax.dot_general(pooled, pooled, (((1,), (1,)), ((), ())),
                             preferred_element_type=jnp.float32)
        r2 = pooled * pooled
        sq_row = _mm_exact(jnp.ones((1, D), jnp.float32), r2,
                           (((1,), (1,)), ((), ())))
        d2 = sq_row - 2.0 * pp

        iota_c = lax.broadcasted_iota(jnp.int32, (G, G), 1)
        out = jnp.full((G, D), -jnp.inf, jnp.float32)
        d2w = d2
        for _ in range(K):
            m = jnp.min(d2w, axis=1, keepdims=True)
            am = jnp.min(jnp.where(d2w == m, iota_c, G), axis=1, keepdims=True)
            sel = (iota_c == am).astype(jnp.float32)
            xj = _mm_exact(sel, pooled)
            cat = jnp.concatenate([pooled, xj - pooled], axis=1)
            msg = _mm(cat, wd_ref[...]) + bd_ref[...]
            out = jnp.maximum(out, jnp.where(msg >= 0.0, msg, 0.01 * msg))
            d2w = jnp.where(iota_c == am, jnp.inf, d2w)
        o_ref[...] = out


def _tc_final(aggp, h1, wrel, brel, wroot, batch2d, wd, bd):
    return pl.pallas_call(
        _final_body,
        grid=(NB,),
        in_specs=[
            pl.BlockSpec((NC, R, D), lambda i: (0, i, 0)),
            pl.BlockSpec((R, D), lambda i: (i, 0)),
            pl.BlockSpec((D, D), lambda i: (0, 0)),
            pl.BlockSpec((1, D), lambda i: (0, 0)),
            pl.BlockSpec((D, D), lambda i: (0, 0)),
            pl.BlockSpec((R, 1), lambda i: (i, 0)),
            pl.BlockSpec((2 * D, D), lambda i: (0, 0)),
            pl.BlockSpec((1, D), lambda i: (0, 0)),
        ],
        out_specs=pl.BlockSpec((G, D), lambda i: (0, 0)),
        out_shape=jax.ShapeDtypeStruct((G, D), jnp.float32),
        scratch_shapes=[
            pltpu.VMEM((G, D), jnp.float32),
            pltpu.VMEM((G, D), jnp.float32),
        ],
    )(aggp, h1, wrel, brel, wroot, batch2d, wd, bd)


def kernel(x, edge_index, batch, W_rel1, b_rel1, W_root1,
           W_rel2, b_rel2, W_root2, Wd, bd):
    src = edge_index[0]
    dst = edge_index[1]
    zero = jnp.zeros((N, D), jnp.float32)
    batch2d = batch.reshape(N, 1)

    sc_segsum = _make_sc_segsum()
    aggp1 = sc_segsum(x, src, dst, zero)
    h1 = _tc_conv1(aggp1, x, W_rel1, b_rel1.reshape(1, D), W_root1)
    aggp2 = sc_segsum(h1, src, dst, zero)
    out = _tc_final(aggp2, h1, W_rel2, b_rel2.reshape(1, D), W_root2,
                    batch2d, Wd, bd.reshape(1, D))
    return out
